# baseline (speedup 1.0000x reference)
import functools

import jax
import jax.numpy as jnp
from jax.experimental import pallas as pl
from jax.experimental.pallas import tpu as pltpu

_BR = 64
_BC = 2048
_NEG = -3.0e38
_LOG2E = 1.4426950408889634
_T = 16.0 * _LOG2E
_A = 1.012 * 16.0 * _LOG2E
_B2 = 0.012 * 16.0 * _LOG2E


def _tree(op, xs):
    xs = list(xs)
    while len(xs) > 1:
        nxt = [op(xs[k], xs[k + 1]) for k in range(0, len(xs) - 1, 2)]
        if len(xs) % 2:
            nxt.append(xs[-1])
        xs = nxt
    return xs[0]


def _loss_body(gt_ref, pt_ref, x_ref, p_ref, out_ref, m_ref, s_ref, o_ref,
               *, last_valid, bc, inv_scale):
    i = pl.program_id(0)
    j = pl.program_id(1)
    nj = pl.num_programs(1)

    @pl.when(j == 0)
    def _init():
        m_ref[...] = jnp.full_like(m_ref, _NEG)
        s_ref[...] = jnp.zeros_like(s_ref)

    gt = gt_ref[...]

    def process(valid):
        n_chunks = (valid + 127) // 128
        m_acc = m_ref[...]
        m_run = m_acc
        for k in range(n_chunks):
            sl = slice(k * 128, (k + 1) * 128)
            xs = x_ref[:, sl]
            ps = p_ref[:, sl]
            cam = jnp.where(ps != 0.0, xs, -10000.0)
            o = jnp.where(cam >= gt, _A * xs + _B2, _T * xs)
            rem = valid - k * 128
            if rem < 128:
                lane = jax.lax.broadcasted_iota(jnp.int32, o.shape, 1)
                o = jnp.where(lane < rem, o, _NEG)
            o_ref[:, sl] = o
            m_run = jnp.maximum(m_run, o)
        s_even = s_ref[...] * jnp.exp2(m_acc - m_run)
        s_odd = jnp.zeros_like(s_even)
        for k in range(n_chunks):
            sl = slice(k * 128, (k + 1) * 128)
            e = jnp.exp2(o_ref[:, sl] - m_run)
            if k % 2 == 0:
                s_even = s_even + e
            else:
                s_odd = s_odd + e
        m_ref[...] = m_run
        s_ref[...] = s_even + s_odd

    if last_valid == bc:
        process(bc)
    else:
        @pl.when(j < nj - 1)
        def _full():
            process(bc)

        @pl.when(j == nj - 1)
        def _last():
            process(last_valid)

    @pl.when(j == nj - 1)
    def _finalize():
        pt = pt_ref[...]
        gt1 = gt[:, :1]
        m_acc = m_ref[...]
        s_acc = s_ref[...]
        m_row = jnp.max(m_acc, axis=1, keepdims=True)
        s_row = jnp.sum(s_acc * jnp.exp2(m_acc - m_row), axis=1,
                        keepdims=True)
        o_true = _T * (gt1 - 0.1)
        cam_t = jnp.where(pt != 0.0, gt1, -10000.0)
        o_gen = jnp.where(cam_t >= gt1, _A * gt1 + _B2, _T * gt1)
        m2 = jnp.maximum(m_row, o_true)
        s2 = (s_row * jnp.exp2(m_row - m2) - jnp.exp2(o_gen - m2)
              + jnp.exp2(o_true - m2))
        lse2 = m2 + jnp.log2(s2)
        out_ref[...] = (jnp.sum(lse2 - o_true) * inv_scale).reshape(1, 1, 1)


def kernel(inputs, targets, mask, pos_cam_mask):
    del mask
    b, c = inputs.shape

    br = min(_BR, b)
    bc = min(_BC, (c + 127) // 128 * 128)
    ni = (b + br - 1) // br
    nj = (c + bc - 1) // bc
    last_valid = c - (nj - 1) * bc

    t2d = targets[:, None]
    gt = jnp.take_along_axis(inputs, t2d, axis=1)
    gtb = jnp.broadcast_to(gt, (b, 128))
    pt = jnp.take_along_axis(pos_cam_mask, t2d, axis=1)

    body = functools.partial(_loss_body, last_valid=last_valid, bc=bc,
                             inv_scale=1.0 / (b * _LOG2E))
    out = pl.pallas_call(
        body,
        grid=(ni, nj),
        in_specs=[
            pl.BlockSpec((br, 128), lambda i, j: (i, 0)),
            pl.BlockSpec((br, 1), lambda i, j: (i, 0)),
            pl.BlockSpec((br, bc), lambda i, j: (i, j)),
            pl.BlockSpec((br, bc), lambda i, j: (i, j)),
        ],
        out_specs=pl.BlockSpec((1, 1, 1), lambda i, j: (i, 0, 0)),
        out_shape=jax.ShapeDtypeStruct((ni, 1, 1), jnp.float32),
        scratch_shapes=[
            pltpu.VMEM((br, 128), jnp.float32),
            pltpu.VMEM((br, 128), jnp.float32),
            pltpu.VMEM((br, bc), jnp.float32),
        ],
        compiler_params=pltpu.CompilerParams(
            dimension_semantics=("parallel", "arbitrary")),
    )(gtb, pt, inputs, pos_cam_mask)
    return jnp.sum(out)


# speedup vs baseline: 1.3585x; 1.3585x over previous
import functools

import jax
import jax.numpy as jnp
from jax.experimental import pallas as pl
from jax.experimental.pallas import tpu as pltpu

_BR = 64
_BC = 2048
_NS = 8
_NEG = -3.0e38
_LOG2E = 1.4426950408889634
_T = 16.0 * _LOG2E
_A = 1.012 * 16.0 * _LOG2E
_B2 = 0.012 * 16.0 * _LOG2E


def _loss_body(*refs, ns, tail_valid, bc, inv_scale):
    n_tail = len(tail_valid)
    n_ops = ns + n_tail
    gt_ref, pt_ref = refs[0], refs[1]
    x_refs = refs[2:2 + n_ops]
    p_refs = refs[2 + n_ops:2 + 2 * n_ops]
    out_ref, m_ref, s_ref, o_ref = refs[2 + 2 * n_ops:]

    j = pl.program_id(1)
    nj = pl.num_programs(1)

    @pl.when(j == 0)
    def _init():
        m_ref[...] = jnp.full_like(m_ref, _NEG)
        s_ref[...] = jnp.zeros_like(s_ref)

    gt = gt_ref[...]

    def process(x_ref, p_ref, valid):
        n_chunks = (valid + 127) // 128
        m_acc = m_ref[...]
        m_run = m_acc
        for k in range(n_chunks):
            sl = slice(k * 128, (k + 1) * 128)
            xs = x_ref[:, sl]
            ps = p_ref[:, sl]
            cam = jnp.where(ps != 0.0, xs, -10000.0)
            o = jnp.where(cam >= gt, _A * xs + _B2, _T * xs)
            rem = valid - k * 128
            if rem < 128:
                lane = jax.lax.broadcasted_iota(jnp.int32, o.shape, 1)
                o = jnp.where(lane < rem, o, _NEG)
            o_ref[:, sl] = o
            m_run = jnp.maximum(m_run, o)
        s_even = s_ref[...] * jnp.exp2(m_acc - m_run)
        s_odd = jnp.zeros_like(s_even)
        for k in range(n_chunks):
            sl = slice(k * 128, (k + 1) * 128)
            e = jnp.exp2(o_ref[:, sl] - m_run)
            if k % 2 == 0:
                s_even = s_even + e
            else:
                s_odd = s_odd + e
        m_ref[...] = m_run
        s_ref[...] = s_even + s_odd

    for g in range(ns):
        process(x_refs[g], p_refs[g], bc)

    if n_tail:
        @pl.when(j == 0)
        def _tail():
            for tdx, tv in enumerate(tail_valid):
                process(x_refs[ns + tdx], p_refs[ns + tdx], tv)

    @pl.when(j == nj - 1)
    def _finalize():
        pt = pt_ref[...]
        gt1 = gt[:, :1]
        m_acc = m_ref[...]
        s_acc = s_ref[...]
        m_row = jnp.max(m_acc, axis=1, keepdims=True)
        s_row = jnp.sum(s_acc * jnp.exp2(m_acc - m_row), axis=1,
                        keepdims=True)
        o_true = _T * (gt1 - 0.1)
        cam_t = jnp.where(pt != 0.0, gt1, -10000.0)
        o_gen = jnp.where(cam_t >= gt1, _A * gt1 + _B2, _T * gt1)
        m2 = jnp.maximum(m_row, o_true)
        s2 = (s_row * jnp.exp2(m_row - m2) - jnp.exp2(o_gen - m2)
              + jnp.exp2(o_true - m2))
        lse2 = m2 + jnp.log2(s2)
        out_ref[...] = (jnp.sum(lse2 - o_true) * inv_scale).reshape(1, 1, 1)


def kernel(inputs, targets, mask, pos_cam_mask):
    del mask
    b, c = inputs.shape

    br = min(_BR, b)
    bc = min(_BC, (c + 127) // 128 * 128)
    ni = (b + br - 1) // br

    n_full = c // bc
    if n_full >= _NS and n_full % _NS == 0:
        ns = _NS
    elif n_full >= 1:
        ns = 1
    else:
        ns = 0
    njh = n_full // ns if ns else 0
    covered = ns * njh * bc
    tail_valid = []
    pos = covered
    while pos < c:
        tail_valid.append(min(bc, c - pos))
        pos += bc

    t2d = targets[:, None]
    gt = jnp.take_along_axis(inputs, t2d, axis=1)
    gtb = jnp.broadcast_to(gt, (b, 128))
    pt = jnp.take_along_axis(pos_cam_mask, t2d, axis=1)

    def group_spec(g):
        return pl.BlockSpec((br, bc), lambda i, j, g=g: (i, j + g * njh))

    def tail_spec(tdx):
        blk = ns * njh + tdx
        return pl.BlockSpec((br, bc), lambda i, j, blk=blk: (i, blk))

    data_specs = ([group_spec(g) for g in range(ns)]
                  + [tail_spec(tdx) for tdx in range(len(tail_valid))])
    in_specs = [
        pl.BlockSpec((br, 128), lambda i, j: (i, 0)),
        pl.BlockSpec((br, 1), lambda i, j: (i, 0)),
    ] + data_specs + data_specs

    body = functools.partial(_loss_body, ns=ns, tail_valid=tuple(tail_valid),
                             bc=bc, inv_scale=1.0 / (b * _LOG2E))
    n_ops = ns + len(tail_valid)
    out = pl.pallas_call(
        body,
        grid=(ni, max(njh, 1)),
        in_specs=in_specs,
        out_specs=pl.BlockSpec((1, 1, 1), lambda i, j: (i, 0, 0)),
        out_shape=jax.ShapeDtypeStruct((ni, 1, 1), jnp.float32),
        scratch_shapes=[
            pltpu.VMEM((br, 128), jnp.float32),
            pltpu.VMEM((br, 128), jnp.float32),
            pltpu.VMEM((br, bc), jnp.float32),
        ],
        compiler_params=pltpu.CompilerParams(
            dimension_semantics=("parallel", "arbitrary")),
    )(gtb, pt, *([inputs] * n_ops + [pos_cam_mask] * n_ops))
    return jnp.sum(out)
